# baseline (speedup 1.0000x reference)
import functools

import jax
import jax.numpy as jnp
from jax import lax
from jax.experimental import pallas as pl
from jax.experimental.pallas import tpu as pltpu
from jax.experimental.pallas import tpu_sc as plsc

EMBED_DIM = 16
BATCH = 16384
NUM_CORES = 2
NUM_SUBCORES = 16
NUM_WORKERS = NUM_CORES * NUM_SUBCORES
BPW = BATCH // NUM_WORKERS
EPS = 1e-5


def _sc_gather_body(idx1_hbm, idx2_hbm, table_hbm, t1_hbm, t2_hbm,
                    idx1_v, idx2_v, rows1_v, rows2_v, sem1, sem2):
    wid = lax.axis_index("s") * NUM_CORES + lax.axis_index("c")
    base = wid * BPW
    pltpu.sync_copy(idx1_hbm.at[pl.ds(base, BPW)], idx1_v)
    pltpu.sync_copy(idx2_hbm.at[pl.ds(base, BPW)], idx2_v)
    cp1 = pltpu.async_copy(table_hbm.at[idx1_v], rows1_v, sem1)
    cp2 = pltpu.async_copy(table_hbm.at[idx2_v], rows2_v, sem2)
    cp1.wait()
    pltpu.sync_copy(rows1_v, t1_hbm.at[pl.ds(base, BPW)])
    cp2.wait()
    pltpu.sync_copy(rows2_v, t2_hbm.at[pl.ds(base, BPW)])


@jax.jit
def _sc_gather(idx1, idx2, table):
    mesh = plsc.VectorSubcoreMesh(core_axis_name="c", subcore_axis_name="s")
    fn = functools.partial(
        pl.kernel,
        mesh=mesh,
        out_type=[
            jax.ShapeDtypeStruct((BATCH, EMBED_DIM), jnp.float32),
            jax.ShapeDtypeStruct((BATCH, EMBED_DIM), jnp.float32),
        ],
        scratch_types=[
            pltpu.VMEM((BPW,), jnp.int32),
            pltpu.VMEM((BPW,), jnp.int32),
            pltpu.VMEM((BPW, EMBED_DIM), jnp.float32),
            pltpu.VMEM((BPW, EMBED_DIM), jnp.float32),
            pltpu.SemaphoreType.DMA,
            pltpu.SemaphoreType.DMA,
        ],
        compiler_params=pltpu.CompilerParams(use_tc_tiling_on_sc=False),
    )(_sc_gather_body)
    return fn(idx1, idx2, table)


def _tc_classifier_body(t1_ref, t2_ref, sd_ref, g1_ref, g2_ref, b1_ref,
                        b2_ref, w1_ref, w2_ref, gsd_ref, bsd_ref, wsd_ref,
                        bias_ref, out_ref):
    t1 = t1_ref[...]
    t2 = t2_ref[...]
    sd = sd_ref[...]
    inv_b = 1.0 / BATCH

    m1 = jnp.sum(t1, axis=0, keepdims=True) * inv_b
    m2 = jnp.sum(t2, axis=0, keepdims=True) * inv_b
    msd = jnp.sum(sd, axis=0, keepdims=True) * inv_b
    c1 = t1 - m1
    c2 = t2 - m2
    csd = sd - msd
    v1 = jnp.sum(c1 * c1, axis=0, keepdims=True) * inv_b
    v2 = jnp.sum(c2 * c2, axis=0, keepdims=True) * inv_b
    vsd = jnp.sum(csd * csd, axis=0, keepdims=True) * inv_b

    s1 = g1_ref[...] * jax.lax.rsqrt(v1 + EPS)
    s2 = g2_ref[...] * jax.lax.rsqrt(v2 + EPS)
    ssd = gsd_ref[...] * jax.lax.rsqrt(vsd + EPS)

    f1 = c1 * s1 + b1_ref[...]
    f2 = c2 * s2 + b2_ref[...]
    fsd = csd * ssd + bsd_ref[...]

    l1 = jax.lax.dot(f1, w1_ref[...], preferred_element_type=jnp.float32)
    l2 = jax.lax.dot(f2, w2_ref[...], preferred_element_type=jnp.float32)
    logits = l1 + l2 + fsd * wsd_ref[...] + bias_ref[...]
    out_ref[...] = 1.0 / (1.0 + jnp.exp(-logits))


@jax.jit
def _tc_classifier(t1, t2, sd, g1, g2, b1, b2, w1, w2, gsd, bsd, wsd, bias):
    return pl.pallas_call(
        _tc_classifier_body,
        out_shape=jax.ShapeDtypeStruct((BATCH, 1), jnp.float32),
    )(t1, t2, sd, g1, g2, b1, b2, w1, w2, gsd, bsd, wsd, bias)


def kernel(idsTensor, table, gamma, beta, W, b):
    idx1 = idsTensor[:, 0].astype(jnp.int32)
    idx2 = idsTensor[:, 1].astype(jnp.int32)
    sd = idsTensor[:, 2:3]
    t1, t2 = _sc_gather(idx1, idx2, table)
    g1 = gamma[:EMBED_DIM].reshape(1, EMBED_DIM)
    g2 = gamma[EMBED_DIM:2 * EMBED_DIM].reshape(1, EMBED_DIM)
    b1 = beta[:EMBED_DIM].reshape(1, EMBED_DIM)
    b2 = beta[EMBED_DIM:2 * EMBED_DIM].reshape(1, EMBED_DIM)
    w1 = W[0, :EMBED_DIM].reshape(EMBED_DIM, 1)
    w2 = W[0, EMBED_DIM:2 * EMBED_DIM].reshape(EMBED_DIM, 1)
    gsd = gamma[2 * EMBED_DIM].reshape(1, 1)
    bsd = beta[2 * EMBED_DIM].reshape(1, 1)
    wsd = W[0, 2 * EMBED_DIM].reshape(1, 1)
    bias = b.reshape(1, 1)
    return _tc_classifier(t1, t2, sd, g1, g2, b1, b2, w1, w2, gsd, bsd, wsd,
                          bias)
